# baseline (speedup 1.0000x reference)
import functools

import jax
import jax.numpy as jnp
from jax import lax
from jax.experimental import pallas as pl
from jax.experimental.pallas import tpu as pltpu
from jax.experimental.pallas import tpu_sc as plsc

V = 10000
F = 128
H = 128
C = 16
E = 320000

NC, NS, L = 2, 16, 16
NW = NC * NS
CHUNK = 128
CH = -(-E // (NW * CHUNK * 2)) * 2
EP = NW * CH * CHUNK
VP = 10240
RPT = VP // NS
PKW = CHUNK // 2

_mesh = plsc.VectorSubcoreMesh(core_axis_name="c", subcore_axis_name="s")
_sc_params = pltpu.CompilerParams(use_tc_tiling_on_sc=False)


def _fill_rows(ref, n, d, val):
    v = jnp.full((L,), val, jnp.float32)

    def body(i, _):
        for j in range(d // L):
            ref[i, pl.ds(j * L, L)] = v
        return 0

    lax.fori_loop(0, n, body, 0)


def _unpack_idx(pk, j, out):
    for k in range(PKW // L):
        w = pk[j, pl.ds(k * L, L)]
        out[pl.ds(k * L, L)] = w & 0xFFFF
        out[pl.ds(PKW + k * L, L)] = (w >> 16) & 0xFFFF


def _make_deg_kernel():
    @functools.partial(
        pl.kernel,
        mesh=_mesh,
        compiler_params=_sc_params,
        out_type=jax.ShapeDtypeStruct((NC, VP, L), jnp.float32),
        scratch_types=[
            pltpu.VMEM((CH, CHUNK), jnp.int32),
            pltpu.VMEM((128, L), jnp.float32),
            pltpu.VMEM((128, L), jnp.float32),
            pltpu.VMEM_SHARED((VP, L), jnp.float32),
        ],
    )
    def deg_kernel(dst_hbm, out_hbm, dstv, onesb, zb, acc):
        c = lax.axis_index("c")
        s = lax.axis_index("s")
        pltpu.sync_copy(dst_hbm.at[c, s], dstv)
        _fill_rows(onesb, 128, L, 1.0)
        _fill_rows(zb, 128, L, 0.0)
        for k in range(RPT // 128):
            pltpu.sync_copy(zb, acc.at[pl.ds(s * RPT + k * 128, 128)])
        plsc.subcore_barrier()

        def body(j, _):
            pltpu.sync_copy(onesb, acc.at[dstv.at[j]], add=True)
            return 0

        lax.fori_loop(0, CH, body, 0)
        plsc.subcore_barrier()
        for k in range(RPT // 128):
            pltpu.sync_copy(acc.at[pl.ds(s * RPT + k * 128, 128)], zb)
            pltpu.sync_copy(zb, out_hbm.at[c, pl.ds(s * RPT + k * 128, 128)])

    return deg_kernel


def _make_edge_kernel(d):
    @functools.partial(
        pl.kernel,
        mesh=_mesh,
        compiler_params=_sc_params,
        out_type=jax.ShapeDtypeStruct((NC, VP, d), jnp.float32),
        scratch_types=[
            pltpu.VMEM((CH, CHUNK), jnp.int32),
            pltpu.VMEM((CH, CHUNK), jnp.int32),
            pltpu.VMEM((CHUNK, d), jnp.float32),
            pltpu.VMEM_SHARED((VP, d), jnp.float32),
            pltpu.SemaphoreType.DMA,
        ],
    )
    def edge_kernel(src_hbm, dst_hbm, g_hbm, out_hbm, srcv, dstv, rows,
                    acc, sem):
        c = lax.axis_index("c")
        s = lax.axis_index("s")
        pltpu.sync_copy(src_hbm.at[c, s], srcv)
        pltpu.sync_copy(dst_hbm.at[c, s], dstv)
        _fill_rows(rows, CHUNK, d, 0.0)
        for k in range(RPT // 128):
            pltpu.sync_copy(rows, acc.at[pl.ds(s * RPT + k * 128, 128)])
        plsc.subcore_barrier()

        def body(j, _):
            pltpu.async_copy(g_hbm.at[srcv.at[j]], rows, sem).wait()
            pltpu.sync_copy(rows, acc.at[dstv.at[j]], add=True)
            return 0

        lax.fori_loop(0, CH, body, 0)
        plsc.subcore_barrier()
        for k in range(RPT // 128):
            pltpu.sync_copy(acc.at[pl.ds(s * RPT + k * 128, 128)], rows)
            pltpu.sync_copy(rows, out_hbm.at[c, pl.ds(s * RPT + k * 128, 128)])

    return edge_kernel


_deg_kernel = _make_deg_kernel()
_edge_kernel_h = _make_edge_kernel(H)
_edge_kernel_c = _make_edge_kernel(C)


_BLK = 512
_GRID = VP // _BLK


def _dinv_block(degp):
    deg = 1.0 + degp[0] + degp[1]
    return lax.rsqrt(deg)


def _tc_b_body(degp, x, w1, g1):
    dinv = _dinv_block(degp[...])
    h = jnp.dot(x[...], w1[...], preferred_element_type=jnp.float32)
    g1[...] = h * jnp.broadcast_to(dinv[:, 0:1], h.shape)


def _tc_d_body(degp, s1, g1, b1, w2, g2):
    dinv = _dinv_block(degp[...])
    blk = g1.shape[0]
    dinvb = jnp.broadcast_to(dinv[:, 0:1], (blk, H))
    pre = dinvb * (s1[0] + s1[1] + g1[...]) + b1[...]
    o = jnp.maximum(pre, 0.0)
    h2 = jnp.dot(o, w2[...], preferred_element_type=jnp.float32)
    g2[...] = h2 * dinv


def _tc_f_body(degp, s2, g2, b2, out):
    dinv = _dinv_block(degp[...])
    out[...] = dinv * (s2[0] + s2[1] + g2[...]) + b2[...]


def _tc_b(degp, x, w1):
    return pl.pallas_call(
        _tc_b_body,
        grid=(_GRID,),
        in_specs=[
            pl.BlockSpec((NC, _BLK, L), lambda i: (0, i, 0)),
            pl.BlockSpec((_BLK, F), lambda i: (i, 0)),
            pl.BlockSpec((F, H), lambda i: (0, 0)),
        ],
        out_specs=pl.BlockSpec((_BLK, H), lambda i: (i, 0)),
        out_shape=jax.ShapeDtypeStruct((VP, H), jnp.float32),
    )(degp, x, w1)


def _tc_d(degp, s1, g1, b1, w2):
    return pl.pallas_call(
        _tc_d_body,
        grid=(_GRID,),
        in_specs=[
            pl.BlockSpec((NC, _BLK, L), lambda i: (0, i, 0)),
            pl.BlockSpec((NC, _BLK, H), lambda i: (0, i, 0)),
            pl.BlockSpec((_BLK, H), lambda i: (i, 0)),
            pl.BlockSpec((1, H), lambda i: (0, 0)),
            pl.BlockSpec((H, C), lambda i: (0, 0)),
        ],
        out_specs=pl.BlockSpec((_BLK, C), lambda i: (i, 0)),
        out_shape=jax.ShapeDtypeStruct((VP, C), jnp.float32),
    )(degp, s1, g1, b1, w2)


def _tc_f(degp, s2, g2, b2):
    return pl.pallas_call(
        _tc_f_body,
        grid=(_GRID,),
        in_specs=[
            pl.BlockSpec((NC, _BLK, L), lambda i: (0, i, 0)),
            pl.BlockSpec((NC, _BLK, C), lambda i: (0, i, 0)),
            pl.BlockSpec((_BLK, C), lambda i: (i, 0)),
            pl.BlockSpec((1, C), lambda i: (0, 0)),
        ],
        out_specs=pl.BlockSpec((_BLK, C), lambda i: (i, 0)),
        out_shape=jax.ShapeDtypeStruct((VP, C), jnp.float32),
    )(degp, s2, g2, b2)


def kernel(x, edge_index, W1, b1, W2, b2):
    src = edge_index[0].astype(jnp.int32)
    dst = edge_index[1].astype(jnp.int32)
    pad = EP - E

    def slab(a):
        a = jnp.concatenate([a, jnp.full((pad,), V, jnp.int32)])
        return a.reshape(NC, NS, CH, CHUNK)

    src_r = slab(src)
    dst_r = slab(dst)
    x_p = jnp.zeros((VP, F), jnp.float32).at[:V].set(x)

    degp = _deg_kernel(dst_r)
    g1 = _tc_b(degp, x_p, W1)
    s1 = _edge_kernel_h(src_r, dst_r, g1)
    g2 = _tc_d(degp, s1, g1, b1.reshape(1, H), W2)
    s2 = _edge_kernel_c(src_r, dst_r, g2)
    out = _tc_f(degp, s2, g2, b2.reshape(1, C))
    return out[:V]


# speedup vs baseline: 1.2501x; 1.2501x over previous
import functools

import jax
import jax.numpy as jnp
from jax import lax
from jax.experimental import pallas as pl
from jax.experimental.pallas import tpu as pltpu
from jax.experimental.pallas import tpu_sc as plsc

V = 10000
F = 128
H = 128
C = 16
E = 320000

NC, NS, L = 2, 16, 16
NW = NC * NS
CHUNK = 128
CH = -(-E // (NW * CHUNK))
EP = NW * CH * CHUNK
VP = 10240
RPT = VP // NS
PKW = CHUNK // 2

_mesh = plsc.VectorSubcoreMesh(core_axis_name="c", subcore_axis_name="s")
_sc_params = pltpu.CompilerParams(use_tc_tiling_on_sc=False)


def _fill_rows(ref, n, d, val):
    v = jnp.full((L,), val, jnp.float32)

    def body(i, _):
        for j in range(d // L):
            ref[i, pl.ds(j * L, L)] = v
        return 0

    lax.fori_loop(0, n, body, 0)


def _unpack_idx(pk, j, out):
    for k in range(PKW // L):
        w = pk[j, pl.ds(k * L, L)]
        out[pl.ds(k * L, L)] = w & 0xFFFF
        out[pl.ds(PKW + k * L, L)] = (w >> 16) & 0xFFFF


def _make_deg_kernel():
    @functools.partial(
        pl.kernel,
        mesh=_mesh,
        compiler_params=_sc_params,
        out_type=jax.ShapeDtypeStruct((NC, VP, L), jnp.float32),
        scratch_types=[
            pltpu.VMEM((CH, CHUNK), jnp.int32),
            pltpu.VMEM((128, L), jnp.float32),
            pltpu.VMEM((128, L), jnp.float32),
            pltpu.VMEM_SHARED((VP, L), jnp.float32),
        ],
    )
    def deg_kernel(dst_hbm, out_hbm, dstv, onesb, zb, acc):
        c = lax.axis_index("c")
        s = lax.axis_index("s")
        pltpu.sync_copy(dst_hbm.at[c, s], dstv)
        _fill_rows(onesb, 128, L, 1.0)
        _fill_rows(zb, 128, L, 0.0)
        for k in range(RPT // 128):
            pltpu.sync_copy(zb, acc.at[pl.ds(s * RPT + k * 128, 128)])
        plsc.subcore_barrier()

        def body(j, _):
            pltpu.sync_copy(onesb, acc.at[dstv.at[j]], add=True)
            return 0

        lax.fori_loop(0, CH, body, 0)
        plsc.subcore_barrier()
        for k in range(RPT // 128):
            pltpu.sync_copy(acc.at[pl.ds(s * RPT + k * 128, 128)], zb)
            pltpu.sync_copy(zb, out_hbm.at[c, pl.ds(s * RPT + k * 128, 128)])

    return deg_kernel


def _make_edge_kernel(d):
    @functools.partial(
        pl.kernel,
        mesh=_mesh,
        compiler_params=_sc_params,
        out_type=jax.ShapeDtypeStruct((NC, VP, d), jnp.float32),
        scratch_types=[
            pltpu.VMEM((CH, CHUNK), jnp.int32),
            pltpu.VMEM((CH, CHUNK), jnp.int32),
            pltpu.VMEM((CHUNK, d), jnp.float32),
            pltpu.VMEM_SHARED((VP, d), jnp.float32),
            pltpu.SemaphoreType.DMA,
        ],
    )
    def edge_kernel(src_hbm, dst_hbm, g_hbm, out_hbm, srcv, dstv, rows,
                    acc, sem):
        c = lax.axis_index("c")
        s = lax.axis_index("s")
        pltpu.sync_copy(src_hbm.at[c, s], srcv)
        pltpu.sync_copy(dst_hbm.at[c, s], dstv)
        _fill_rows(rows, CHUNK, d, 0.0)
        for k in range(RPT // 128):
            pltpu.sync_copy(rows, acc.at[pl.ds(s * RPT + k * 128, 128)])
        plsc.subcore_barrier()

        def body(j, _):
            pltpu.async_copy(g_hbm.at[srcv.at[j]], rows, sem).wait()
            pltpu.sync_copy(rows, acc.at[dstv.at[j]], add=True)
            return 0

        lax.fori_loop(0, CH, body, 0)
        plsc.subcore_barrier()
        for k in range(RPT // 128):
            pltpu.sync_copy(acc.at[pl.ds(s * RPT + k * 128, 128)], rows)
            pltpu.sync_copy(rows, out_hbm.at[c, pl.ds(s * RPT + k * 128, 128)])

    return edge_kernel


_deg_kernel = _make_deg_kernel()
_edge_kernel_h = _make_edge_kernel(H)
_edge_kernel_c = _make_edge_kernel(C)


_BLK = 512
_GRID = VP // _BLK


def _dinv_block(degp):
    deg = 1.0 + degp[0] + degp[1]
    return lax.rsqrt(deg)


def _tc_b_body(degp, x, w1, g1):
    dinv = _dinv_block(degp[...])
    h = jnp.dot(x[...], w1[...], preferred_element_type=jnp.float32)
    g1[...] = h * jnp.broadcast_to(dinv[:, 0:1], h.shape)


def _tc_d_body(degp, s1, g1, b1, w2, g2):
    dinv = _dinv_block(degp[...])
    blk = g1.shape[0]
    dinvb = jnp.broadcast_to(dinv[:, 0:1], (blk, H))
    pre = dinvb * (s1[0] + s1[1] + g1[...]) + b1[...]
    o = jnp.maximum(pre, 0.0)
    h2 = jnp.dot(o, w2[...], preferred_element_type=jnp.float32)
    g2[...] = h2 * dinv


def _tc_f_body(degp, s2, g2, b2, out):
    dinv = _dinv_block(degp[...])
    out[...] = dinv * (s2[0] + s2[1] + g2[...]) + b2[...]


def _tc_b(degp, x, w1):
    return pl.pallas_call(
        _tc_b_body,
        grid=(_GRID,),
        in_specs=[
            pl.BlockSpec((NC, _BLK, L), lambda i: (0, i, 0)),
            pl.BlockSpec((_BLK, F), lambda i: (i, 0)),
            pl.BlockSpec((F, H), lambda i: (0, 0)),
        ],
        out_specs=pl.BlockSpec((_BLK, H), lambda i: (i, 0)),
        out_shape=jax.ShapeDtypeStruct((VP, H), jnp.float32),
    )(degp, x, w1)


def _tc_d(degp, s1, g1, b1, w2):
    return pl.pallas_call(
        _tc_d_body,
        grid=(_GRID,),
        in_specs=[
            pl.BlockSpec((NC, _BLK, L), lambda i: (0, i, 0)),
            pl.BlockSpec((NC, _BLK, H), lambda i: (0, i, 0)),
            pl.BlockSpec((_BLK, H), lambda i: (i, 0)),
            pl.BlockSpec((1, H), lambda i: (0, 0)),
            pl.BlockSpec((H, C), lambda i: (0, 0)),
        ],
        out_specs=pl.BlockSpec((_BLK, C), lambda i: (i, 0)),
        out_shape=jax.ShapeDtypeStruct((VP, C), jnp.float32),
    )(degp, s1, g1, b1, w2)


def _tc_f(degp, s2, g2, b2):
    return pl.pallas_call(
        _tc_f_body,
        grid=(_GRID,),
        in_specs=[
            pl.BlockSpec((NC, _BLK, L), lambda i: (0, i, 0)),
            pl.BlockSpec((NC, _BLK, C), lambda i: (0, i, 0)),
            pl.BlockSpec((_BLK, C), lambda i: (i, 0)),
            pl.BlockSpec((1, C), lambda i: (0, 0)),
        ],
        out_specs=pl.BlockSpec((_BLK, C), lambda i: (i, 0)),
        out_shape=jax.ShapeDtypeStruct((VP, C), jnp.float32),
    )(degp, s2, g2, b2)


def kernel(x, edge_index, W1, b1, W2, b2):
    src = edge_index[0].astype(jnp.int32)
    dst = edge_index[1].astype(jnp.int32)
    pad = EP - E

    def slab(a, fill):
        a = jnp.concatenate([a, fill])
        return a.reshape(NC, NS, CH, CHUNK)

    src_r = slab(src, jnp.full((pad,), V, jnp.int32))
    dst_r = slab(dst, V + (jnp.arange(pad, dtype=jnp.int32) % (VP - V)))
    x_p = jnp.zeros((VP, F), jnp.float32).at[:V].set(x)

    degp = _deg_kernel(dst_r)
    g1 = _tc_b(degp, x_p, W1)
    s1 = _edge_kernel_h(src_r, dst_r, g1)
    g2 = _tc_d(degp, s1, g1, b1.reshape(1, H), W2)
    s2 = _edge_kernel_c(src_r, dst_r, g2)
    out = _tc_f(degp, s2, g2, b2.reshape(1, C))
    return out[:V]


# speedup vs baseline: 1.3377x; 1.0701x over previous
import functools

import jax
import jax.numpy as jnp
from jax import lax
from jax.experimental import pallas as pl
from jax.experimental.pallas import tpu as pltpu
from jax.experimental.pallas import tpu_sc as plsc

V = 10000
F = 128
H = 128
C = 16
E = 320000

NC, NS, L = 2, 16, 16
NW = NC * NS
CHUNK = 128
CH = -(-E // (NW * CHUNK))
EP = NW * CH * CHUNK
VP = 10240
RPT = VP // NS
PKW = CHUNK // 2

_mesh = plsc.VectorSubcoreMesh(core_axis_name="c", subcore_axis_name="s")
_sc_params = pltpu.CompilerParams(use_tc_tiling_on_sc=False)


def _fill_rows(ref, n, d, val):
    v = jnp.full((L,), val, jnp.float32)

    def body(i, _):
        for j in range(d // L):
            ref[i, pl.ds(j * L, L)] = v
        return 0

    lax.fori_loop(0, n, body, 0)


def _unpack_idx(pk, j, out):
    for k in range(PKW // L):
        w = pk[j, pl.ds(k * L, L)]
        out[pl.ds(k * L, L)] = w & 0xFFFF
        out[pl.ds(PKW + k * L, L)] = (w >> 16) & 0xFFFF


def _make_deg_kernel():
    @functools.partial(
        pl.kernel,
        mesh=_mesh,
        compiler_params=_sc_params,
        out_type=jax.ShapeDtypeStruct((NC, VP, L), jnp.float32),
        scratch_types=[
            pltpu.VMEM((CH, CHUNK), jnp.int32),
            pltpu.VMEM((128, L), jnp.float32),
            pltpu.VMEM((128, L), jnp.float32),
            pltpu.VMEM_SHARED((VP, L), jnp.float32),
        ],
    )
    def deg_kernel(dst_hbm, out_hbm, dstv, onesb, zb, acc):
        c = lax.axis_index("c")
        s = lax.axis_index("s")
        pltpu.sync_copy(dst_hbm.at[c, s], dstv)
        _fill_rows(onesb, 128, L, 1.0)
        _fill_rows(zb, 128, L, 0.0)
        for k in range(RPT // 128):
            pltpu.sync_copy(zb, acc.at[pl.ds(s * RPT + k * 128, 128)])
        plsc.subcore_barrier()

        def body(j, _):
            pltpu.sync_copy(onesb, acc.at[dstv.at[j]], add=True)
            return 0

        lax.fori_loop(0, CH, body, 0)
        plsc.subcore_barrier()
        for k in range(RPT // 128):
            pltpu.sync_copy(acc.at[pl.ds(s * RPT + k * 128, 128)], zb)
            pltpu.sync_copy(zb, out_hbm.at[c, pl.ds(s * RPT + k * 128, 128)])

    return deg_kernel


def _make_edge_kernel(d):
    @functools.partial(
        pl.kernel,
        mesh=_mesh,
        compiler_params=_sc_params,
        out_type=jax.ShapeDtypeStruct((NC, VP, d), jnp.float32),
        scratch_types=[
            pltpu.VMEM((CH, CHUNK), jnp.int32),
            pltpu.VMEM((CH, CHUNK), jnp.int32),
            pltpu.VMEM((CHUNK, d), jnp.float32),
            pltpu.VMEM_SHARED((VP, d), jnp.float32),
            pltpu.SemaphoreType.DMA,
        ],
    )
    def edge_kernel(src_hbm, dst_hbm, g_hbm, out_hbm, srcv, dstv, rows,
                    acc, sem):
        c = lax.axis_index("c")
        s = lax.axis_index("s")
        pltpu.sync_copy(src_hbm.at[c, s], srcv)
        pltpu.sync_copy(dst_hbm.at[c, s], dstv)
        _fill_rows(rows, CHUNK, d, 0.0)
        for k in range(RPT // 128):
            pltpu.sync_copy(rows, acc.at[pl.ds(s * RPT + k * 128, 128)])
        plsc.subcore_barrier()

        def body(j, _):
            pltpu.async_copy(g_hbm.at[srcv.at[j]], rows, sem).wait()
            pltpu.sync_copy(rows, acc.at[dstv.at[j]], add=True)
            return 0

        lax.fori_loop(0, CH, body, 0)
        plsc.subcore_barrier()
        for k in range(RPT // 128):
            pltpu.sync_copy(acc.at[pl.ds(s * RPT + k * 128, 128)], rows)
            pltpu.sync_copy(rows, out_hbm.at[c, pl.ds(s * RPT + k * 128, 128)])

    return edge_kernel


_deg_kernel = _make_deg_kernel()
_edge_kernel_h = _make_edge_kernel(H)
_edge_kernel_c = _make_edge_kernel(C)


_BLK = 512
_GRID = VP // _BLK


def _dinv_block(degp):
    deg = 1.0 + degp[0] + degp[1]
    return lax.rsqrt(deg)


def _tc_b0_body(x, w1, h1):
    h1[...] = jnp.dot(x[...], w1[...], preferred_element_type=jnp.float32)


def _tc_b_body(degp, h, g1):
    dinv = _dinv_block(degp[...])
    g1[...] = h[...] * jnp.broadcast_to(dinv[:, 0:1], h.shape)


def _tc_d_body(degp, s1, g1, b1, w2, g2):
    dinv = _dinv_block(degp[...])
    blk = g1.shape[0]
    dinvb = jnp.broadcast_to(dinv[:, 0:1], (blk, H))
    pre = dinvb * (s1[0] + s1[1] + g1[...]) + b1[...]
    o = jnp.maximum(pre, 0.0)
    h2 = jnp.dot(o, w2[...], preferred_element_type=jnp.float32)
    g2[...] = h2 * dinv


def _tc_f_body(degp, s2, g2, b2, out):
    dinv = _dinv_block(degp[...])
    out[...] = dinv * (s2[0] + s2[1] + g2[...]) + b2[...]


def _tc_b0(x, w1):
    return pl.pallas_call(
        _tc_b0_body,
        grid=(_GRID,),
        in_specs=[
            pl.BlockSpec((_BLK, F), lambda i: (i, 0)),
            pl.BlockSpec((F, H), lambda i: (0, 0)),
        ],
        out_specs=pl.BlockSpec((_BLK, H), lambda i: (i, 0)),
        out_shape=jax.ShapeDtypeStruct((VP, H), jnp.float32),
    )(x, w1)


def _tc_b(degp, h):
    return pl.pallas_call(
        _tc_b_body,
        grid=(_GRID,),
        in_specs=[
            pl.BlockSpec((NC, _BLK, L), lambda i: (0, i, 0)),
            pl.BlockSpec((_BLK, H), lambda i: (i, 0)),
        ],
        out_specs=pl.BlockSpec((_BLK, H), lambda i: (i, 0)),
        out_shape=jax.ShapeDtypeStruct((VP, H), jnp.float32),
    )(degp, h)


def _tc_d(degp, s1, g1, b1, w2):
    return pl.pallas_call(
        _tc_d_body,
        grid=(_GRID,),
        in_specs=[
            pl.BlockSpec((NC, _BLK, L), lambda i: (0, i, 0)),
            pl.BlockSpec((NC, _BLK, H), lambda i: (0, i, 0)),
            pl.BlockSpec((_BLK, H), lambda i: (i, 0)),
            pl.BlockSpec((1, H), lambda i: (0, 0)),
            pl.BlockSpec((H, C), lambda i: (0, 0)),
        ],
        out_specs=pl.BlockSpec((_BLK, C), lambda i: (i, 0)),
        out_shape=jax.ShapeDtypeStruct((VP, C), jnp.float32),
    )(degp, s1, g1, b1, w2)


def _tc_f(degp, s2, g2, b2):
    return pl.pallas_call(
        _tc_f_body,
        grid=(_GRID,),
        in_specs=[
            pl.BlockSpec((NC, _BLK, L), lambda i: (0, i, 0)),
            pl.BlockSpec((NC, _BLK, C), lambda i: (0, i, 0)),
            pl.BlockSpec((_BLK, C), lambda i: (i, 0)),
            pl.BlockSpec((1, C), lambda i: (0, 0)),
        ],
        out_specs=pl.BlockSpec((_BLK, C), lambda i: (i, 0)),
        out_shape=jax.ShapeDtypeStruct((VP, C), jnp.float32),
    )(degp, s2, g2, b2)


def kernel(x, edge_index, W1, b1, W2, b2):
    src = edge_index[0].astype(jnp.int32)
    dst = edge_index[1].astype(jnp.int32)
    pad = EP - E

    def slab(a, fill):
        a = jnp.concatenate([a, fill])
        return a.reshape(NC, NS, CH, CHUNK)

    src_r = slab(src, jnp.full((pad,), V, jnp.int32))
    dst_r = slab(dst, V + (jnp.arange(pad, dtype=jnp.int32) % (VP - V)))
    x_p = jnp.zeros((VP, F), jnp.float32).at[:V].set(x)

    h1 = _tc_b0(x_p, W1)
    degp = _deg_kernel(dst_r)
    g1 = _tc_b(degp, h1)
    s1 = _edge_kernel_h(src_r, dst_r, g1)
    g2 = _tc_d(degp, s1, g1, b1.reshape(1, H), W2)
    s2 = _edge_kernel_c(src_r, dst_r, g2)
    out = _tc_f(degp, s2, g2, b2.reshape(1, C))
    return out[:V]
